# baseline (speedup 1.0000x reference)
import functools

import jax
import jax.numpy as jnp
from jax import lax
from jax.experimental import pallas as pl
from jax.experimental.pallas import tpu as pltpu
from jax.experimental.pallas import tpu_sc as plsc

DIM = 128
MARGIN = 1.0
LANES = 16


def _trans_e_body(idx_hbm, emb_hbm, out_hbm, idx_v, rows_v, out_v, sem):
    is_lead = (lax.axis_index("c") == 0) & (lax.axis_index("s") == 0)

    @pl.when(is_lead)
    def _():
        pltpu.sync_copy(idx_hbm, idx_v)
        pltpu.async_copy(emb_hbm.at[idx_v], rows_v, sem).wait()

        acc = jnp.zeros((LANES,), jnp.float32)
        for j in range(DIM // LANES):
            s = pl.ds(j * LANES, LANES)
            pos = jnp.abs(rows_v[0, s] + rows_v[1, s] - rows_v[2, s])
            neg = jnp.abs(rows_v[3, s] + rows_v[4, s] - rows_v[5, s])
            acc = acc + (pos - neg)
        lanes = lax.iota(jnp.int32, LANES)
        for shift in (8, 4, 2, 1):
            perm = lax.rem(lanes + shift, LANES)
            acc = acc + acc.at[perm].get(mode="promise_in_bounds")
        out_v[...] = jnp.maximum(acc + MARGIN, 0.0)
        pltpu.sync_copy(out_v, out_hbm)


@jax.jit
def _trans_e_loss(idx, embeddings):
    mesh = plsc.VectorSubcoreMesh(core_axis_name="c", subcore_axis_name="s")
    k = functools.partial(
        pl.kernel,
        out_type=jax.ShapeDtypeStruct((LANES,), jnp.float32),
        mesh=mesh,
        scratch_types=[
            pltpu.VMEM((8,), jnp.int32),
            pltpu.VMEM((8, DIM), jnp.float32),
            pltpu.VMEM((LANES,), jnp.float32),
            pltpu.SemaphoreType.DMA,
        ],
    )(_trans_e_body)
    return k(idx, embeddings)[0]


def kernel(pos_exmpl, neg_exmpl, embeddings):
    idx = jnp.concatenate(
        [
            pos_exmpl.astype(jnp.int32),
            neg_exmpl.astype(jnp.int32),
            jnp.zeros((2,), jnp.int32),
        ]
    )
    return _trans_e_loss(idx, embeddings)
